# baseline (speedup 1.0000x reference)
import functools

import jax
import jax.numpy as jnp
from jax import lax
from jax.experimental import pallas as pl
from jax.experimental.pallas import tpu as pltpu
from jax.experimental.pallas import tpu_sc as plsc

ALPHA = 0.1
GW = 160
BN = 512

_HIMASK = -65536


def _pack_bf16_pair(r, d):
    h = d // 2
    lo = lax.bitcast_convert_type(r[:, :h], jnp.int32)
    hi = lax.bitcast_convert_type(r[:, h:], jnp.int32)
    lo = lax.shift_right_logical(lo + 0x8000, 16)
    hi = (hi + 0x8000) & _HIMASK
    return lo | hi


def _unpack_bf16_pair(w):
    hi = lax.bitcast_convert_type(w, jnp.float32)
    lo = lax.bitcast_convert_type(lax.shift_left(w, 16), jnp.float32)
    return lo, hi


def _sc_gather(table, idx_flat, total, h):
    mesh = plsc.VectorSubcoreMesh(core_axis_name="c", subcore_axis_name="s")
    nbuf = 4
    nfly = 2
    chunk = total // 32
    nw = chunk // GW
    assert chunk % GW == 0 and nw % nbuf == 0

    @functools.partial(
        pl.kernel,
        out_type=jax.ShapeDtypeStruct((total, h), table.dtype),
        mesh=mesh,
        scratch_types=[
            pltpu.VMEM((chunk,), jnp.int32),
            pltpu.VMEM((nbuf, GW, h), jnp.int32),
            pltpu.SemaphoreType.DMA((nbuf,)),
            pltpu.SemaphoreType.DMA((nbuf,)),
        ],
    )
    def k(table_hbm, idx_hbm, out_hbm, idx_v, bufs, sem_g, sem_w):
        wid = lax.axis_index("s") * 2 + lax.axis_index("c")
        base = wid * chunk
        pltpu.sync_copy(idx_hbm.at[0, pl.ds(base, chunk)], idx_v)

        def gather(g, b):
            return pltpu.make_async_copy(
                table_hbm.at[idx_v.at[pl.ds(g * GW, GW)]],
                bufs.at[b], sem_g.at[b])

        def writeback(g, b):
            return pltpu.make_async_copy(
                bufs.at[b], out_hbm.at[pl.ds(base + g * GW, GW)], sem_w.at[b])

        for b in range(nfly):
            gather(b, b).start()

        @pl.loop(0, nw, step=nbuf)
        def _(g0):
            for b in range(nbuf):
                g = g0 + b
                gather(g, b).wait()
                writeback(g, b).start()
                b2 = (b + nfly) % nbuf

                @pl.when(g + nfly < nw)
                def _():
                    @pl.when(g >= nbuf - nfly)
                    def _():
                        writeback(g - (nbuf - nfly), b2).wait()

                    gather(g + nfly, b2).start()

        for b in range(nbuf):
            writeback(nw - nbuf + b, b).wait()

    return k(table, idx_flat)


def _tc_in(x, w_t, b, npad, d):
    n = x.shape[0]
    grid = (npad // BN,)

    def body(x_ref, w_ref, b_ref, o_ref, ot_ref):
        i = pl.program_id(0)
        acc = jnp.dot(x_ref[...], w_ref[...],
                      preferred_element_type=jnp.float32,
                      precision=lax.Precision.HIGHEST)
        r = jnp.maximum(acc + b_ref[...], 0.0)
        row = i * BN + lax.broadcasted_iota(jnp.int32, r.shape, 0)
        r = jnp.where(row < n, r, 0.0)
        o_ref[...] = r
        ot_ref[...] = _pack_bf16_pair(r, d)

    xp = jnp.pad(x, ((0, npad - n), (0, 0)))
    return pl.pallas_call(
        body,
        grid=grid,
        in_specs=[
            pl.BlockSpec((BN, x.shape[1]), lambda i: (i, 0)),
            pl.BlockSpec((x.shape[1], d), lambda i: (0, 0)),
            pl.BlockSpec((1, d), lambda i: (0, 0)),
        ],
        out_specs=[
            pl.BlockSpec((BN, d), lambda i: (i, 0)),
            pl.BlockSpec((BN, d // 2), lambda i: (i, 0)),
        ],
        out_shape=[
            jax.ShapeDtypeStruct((npad, d), jnp.float32),
            jax.ShapeDtypeStruct((npad, d // 2), jnp.int32),
        ],
    )(xp, w_t, b.reshape(1, d))


def _tc_layer(g, in_feats, pw, fc_t, nrows, d, k16, npl, row0):
    grid = (nrows // BN,)
    h = d // 2
    blk0 = row0 // BN

    def body(g_ref, f_ref, pw_ref, fc_ref, o_ref):
        lo, hi = _unpack_bf16_pair(g_ref[...])
        np_ = k16 // npl
        lo4 = lo.reshape(np_, npl, BN, h).sum(axis=0)
        hi4 = hi.reshape(np_, npl, BN, h).sum(axis=0)
        pwv = pw_ref[...]
        acc_lo = jnp.sum(lo4 * pwv[:, None, :h], axis=0)
        acc_hi = jnp.sum(hi4 * pwv[:, None, h:], axis=0)
        acc = jnp.concatenate([acc_lo, acc_hi], axis=1)
        r = jnp.dot(acc, fc_ref[...],
                    preferred_element_type=jnp.float32)
        r = jnp.maximum(r, 0.0)
        feats = ALPHA * f_ref[...] + (1.0 - ALPHA) * r
        o_ref[...] = _pack_bf16_pair(feats, d)

    return pl.pallas_call(
        body,
        grid=grid,
        in_specs=[
            pl.BlockSpec((k16, BN, h), lambda i: (0, i, 0)),
            pl.BlockSpec((BN, d), lambda i: (i + blk0, 0)),
            pl.BlockSpec((npl, d), lambda i: (0, 0)),
            pl.BlockSpec((d, d), lambda i: (0, 0)),
        ],
        out_specs=pl.BlockSpec((BN, h), lambda i: (i, 0)),
        out_shape=jax.ShapeDtypeStruct((nrows, h), jnp.int32),
    )(g, in_feats, pw, fc_t)


def _tc_out(table, w_t, b, npad, d, d_out):
    grid = (npad // BN,)
    h = d // 2

    def body(t_ref, w_ref, b_ref, o_ref):
        lo, hi = _unpack_bf16_pair(t_ref[...])
        feats = jnp.concatenate([lo, hi], axis=1)
        acc = jnp.dot(feats, w_ref[...],
                      preferred_element_type=jnp.float32)
        o_ref[...] = acc + b_ref[...]

    return pl.pallas_call(
        body,
        grid=grid,
        in_specs=[
            pl.BlockSpec((BN, h), lambda i: (i, 0)),
            pl.BlockSpec((d, d_out), lambda i: (0, 0)),
            pl.BlockSpec((1, d_out), lambda i: (0, 0)),
        ],
        out_specs=pl.BlockSpec((BN, d_out), lambda i: (i, 0)),
        out_shape=jax.ShapeDtypeStruct((npad, d_out), jnp.float32),
    )(table, w_t, b.reshape(1, d_out))


def kernel(input_x, paths, W_in, b_in, W_out, b_out, path_weight, fc_w):
    n, in_dim = input_x.shape
    p, _, pl_len = paths.shape
    hidden = W_in.shape[0]
    out_dim = W_out.shape[0]
    num_layers = fc_w.shape[0]
    k16 = p * pl_len

    npad = ((n + 1279) // 1280) * 1280
    assert npad % BN == 0
    nchunk = 1
    half = npad // nchunk
    thalf = k16 * half
    assert thalf % (GW * 32) == 0 and half % BN == 0

    idx = paths.transpose(0, 2, 1).reshape(k16, n)
    idx = jnp.pad(idx, ((0, 0), (0, npad - n)))

    pw_all = path_weight[:, 0, :, :] / p

    idx_halves = [idx[:, c * half:(c + 1) * half].reshape(1, thalf)
                  for c in range(nchunk)]

    in_feats, table = _tc_in(input_x, W_in.T, b_in, npad, hidden)
    for l in range(num_layers):
        parts = []
        for c in range(nchunk):
            g = _sc_gather(table, idx_halves[c], thalf, hidden // 2)
            g = g.reshape(k16, half, hidden // 2)
            parts.append(_tc_layer(g, in_feats, pw_all[l], fc_w[l].T, half,
                                   hidden, k16, pl_len, c * half))
        table = jnp.concatenate(parts, axis=0)
    out = _tc_out(table, W_out.T, b_out, npad, hidden, out_dim)
    return out[:n]


# speedup vs baseline: 1.9925x; 1.9925x over previous
import functools

import jax
import jax.numpy as jnp
from jax import lax
from jax.experimental import pallas as pl
from jax.experimental.pallas import tpu as pltpu
from jax.experimental.pallas import tpu_sc as plsc

ALPHA = 0.1
GW = 128
BN = 512

_HIMASK = -65536


def _pack_bf16_pair(r, d):
    h = d // 2
    lo = lax.bitcast_convert_type(r[:, :h], jnp.int32)
    hi = lax.bitcast_convert_type(r[:, h:], jnp.int32)
    lo = lax.shift_right_logical(lo + 0x8000, 16)
    hi = (hi + 0x8000) & _HIMASK
    return lo | hi


def _unpack_bf16_pair(w):
    hi = lax.bitcast_convert_type(w, jnp.float32)
    lo = lax.bitcast_convert_type(lax.shift_left(w, 16), jnp.float32)
    return lo, hi


def _sc_gather(table, idx_flat, total, h):
    mesh = plsc.VectorSubcoreMesh(core_axis_name="c", subcore_axis_name="s")
    nbuf = 4
    nfly = 2
    chunk = total // 32
    nw = chunk // GW
    assert chunk % GW == 0 and nw % nbuf == 0

    @functools.partial(
        pl.kernel,
        out_type=jax.ShapeDtypeStruct((total, h), table.dtype),
        mesh=mesh,
        scratch_types=[
            pltpu.VMEM((chunk,), jnp.int32),
            pltpu.VMEM((nbuf, GW, h), jnp.int32),
            pltpu.SemaphoreType.DMA((nbuf,)),
            pltpu.SemaphoreType.DMA((nbuf,)),
        ],
    )
    def k(table_hbm, idx_hbm, out_hbm, idx_v, bufs, sem_g, sem_w):
        wid = lax.axis_index("s") * 2 + lax.axis_index("c")
        base = wid * chunk
        pltpu.sync_copy(idx_hbm.at[0, pl.ds(base, chunk)], idx_v)

        def gather(g, b):
            return pltpu.make_async_copy(
                table_hbm.at[idx_v.at[pl.ds(g * GW, GW)]],
                bufs.at[b], sem_g.at[b])

        def writeback(g, b):
            return pltpu.make_async_copy(
                bufs.at[b], out_hbm.at[pl.ds(base + g * GW, GW)], sem_w.at[b])

        for b in range(nfly):
            gather(b, b).start()

        @pl.loop(0, nw, step=nbuf)
        def _(g0):
            for b in range(nbuf):
                g = g0 + b
                gather(g, b).wait()
                writeback(g, b).start()
                b2 = (b + nfly) % nbuf

                @pl.when(g + nfly < nw)
                def _():
                    @pl.when(g >= nbuf - nfly)
                    def _():
                        writeback(g - (nbuf - nfly), b2).wait()

                    gather(g + nfly, b2).start()

        for b in range(nbuf):
            writeback(nw - nbuf + b, b).wait()

    return k(table, idx_flat)


def _tc_in(x, w_t, b, npad, d):
    n = x.shape[0]
    grid = (npad // BN,)

    def body(x_ref, w_ref, b_ref, o_ref, ot_ref):
        i = pl.program_id(0)
        acc = jnp.dot(x_ref[...], w_ref[...],
                      preferred_element_type=jnp.float32,
                      precision=lax.Precision.HIGHEST)
        r = jnp.maximum(acc + b_ref[...], 0.0)
        row = i * BN + lax.broadcasted_iota(jnp.int32, r.shape, 0)
        r = jnp.where(row < n, r, 0.0)
        o_ref[...] = r
        ot_ref[...] = _pack_bf16_pair(r, d)

    xp = jnp.pad(x, ((0, npad - n), (0, 0)))
    return pl.pallas_call(
        body,
        grid=grid,
        in_specs=[
            pl.BlockSpec((BN, x.shape[1]), lambda i: (i, 0)),
            pl.BlockSpec((x.shape[1], d), lambda i: (0, 0)),
            pl.BlockSpec((1, d), lambda i: (0, 0)),
        ],
        out_specs=[
            pl.BlockSpec((BN, d), lambda i: (i, 0)),
            pl.BlockSpec((BN, d // 2), lambda i: (i, 0)),
        ],
        out_shape=[
            jax.ShapeDtypeStruct((npad, d), jnp.float32),
            jax.ShapeDtypeStruct((npad, d // 2), jnp.int32),
        ],
    )(xp, w_t, b.reshape(1, d))


def _tc_layer(g, in_feats, pw, fc_t, nrows, d, k16, npl, row0):
    grid = (nrows // BN,)
    h = d // 2
    blk0 = row0 // BN

    def body(g_ref, f_ref, pw_ref, fc_ref, o_ref):
        lo, hi = _unpack_bf16_pair(g_ref[...])
        np_ = k16 // npl
        lo4 = lo.reshape(np_, npl, BN, h).sum(axis=0)
        hi4 = hi.reshape(np_, npl, BN, h).sum(axis=0)
        pwv = pw_ref[...]
        acc_lo = jnp.sum(lo4 * pwv[:, None, :h], axis=0)
        acc_hi = jnp.sum(hi4 * pwv[:, None, h:], axis=0)
        r = (jnp.dot(acc_lo, fc_ref[:h, :], preferred_element_type=jnp.float32)
             + jnp.dot(acc_hi, fc_ref[h:, :], preferred_element_type=jnp.float32))
        r = jnp.maximum(r, 0.0)
        feats = ALPHA * f_ref[...] + (1.0 - ALPHA) * r
        o_ref[...] = _pack_bf16_pair(feats, d)

    return pl.pallas_call(
        body,
        grid=grid,
        in_specs=[
            pl.BlockSpec((k16, BN, h), lambda i: (0, i, 0)),
            pl.BlockSpec((BN, d), lambda i: (i + blk0, 0)),
            pl.BlockSpec((npl, d), lambda i: (0, 0)),
            pl.BlockSpec((d, d), lambda i: (0, 0)),
        ],
        out_specs=pl.BlockSpec((BN, h), lambda i: (i, 0)),
        out_shape=jax.ShapeDtypeStruct((nrows, h), jnp.int32),
    )(g, in_feats, pw, fc_t)


def _tc_out(table, w_t, b, npad, d, d_out):
    grid = (npad // BN,)
    h = d // 2

    def body(t_ref, w_ref, b_ref, o_ref):
        lo, hi = _unpack_bf16_pair(t_ref[...])
        acc = (jnp.dot(lo, w_ref[:h, :], preferred_element_type=jnp.float32)
               + jnp.dot(hi, w_ref[h:, :], preferred_element_type=jnp.float32))
        o_ref[...] = acc + b_ref[...]

    return pl.pallas_call(
        body,
        grid=grid,
        in_specs=[
            pl.BlockSpec((BN, h), lambda i: (i, 0)),
            pl.BlockSpec((d, d_out), lambda i: (0, 0)),
            pl.BlockSpec((1, d_out), lambda i: (0, 0)),
        ],
        out_specs=pl.BlockSpec((BN, d_out), lambda i: (i, 0)),
        out_shape=jax.ShapeDtypeStruct((npad, d_out), jnp.float32),
    )(table, w_t, b.reshape(1, d_out))


def kernel(input_x, paths, W_in, b_in, W_out, b_out, path_weight, fc_w):
    n, in_dim = input_x.shape
    p, _, pl_len = paths.shape
    hidden = W_in.shape[0]
    out_dim = W_out.shape[0]
    num_layers = fc_w.shape[0]
    k16 = p * pl_len

    npad = ((n + 1023) // 1024) * 1024
    assert npad % BN == 0
    nchunk = 1
    half = npad // nchunk
    thalf = k16 * half
    assert thalf % (GW * 32) == 0 and half % BN == 0

    idx = paths.transpose(0, 2, 1).reshape(k16, n)
    idx = jnp.pad(idx, ((0, 0), (0, npad - n)))

    pw_all = path_weight[:, 0, :, :] / p

    idx_halves = [idx[:, c * half:(c + 1) * half].reshape(1, thalf)
                  for c in range(nchunk)]

    in_feats, table = _tc_in(input_x, W_in.T, b_in, npad, hidden)
    for l in range(num_layers):
        parts = []
        for c in range(nchunk):
            g = _sc_gather(table, idx_halves[c], thalf, hidden // 2)
            g = g.reshape(k16, half, hidden // 2)
            parts.append(_tc_layer(g, in_feats, pw_all[l], fc_w[l].T, half,
                                   hidden, k16, pl_len, c * half))
        table = jnp.concatenate(parts, axis=0)
    out = _tc_out(table, W_out.T, b_out, npad, hidden, out_dim)
    return out[:n]


# speedup vs baseline: 1.9954x; 1.0014x over previous
import functools

import jax
import jax.numpy as jnp
from jax import lax
from jax.experimental import pallas as pl
from jax.experimental.pallas import tpu as pltpu
from jax.experimental.pallas import tpu_sc as plsc

ALPHA = 0.1
GW = 128
BN = 512

_HIMASK = -65536


def _pack_bf16_pair(r, d):
    h = d // 2
    lo = lax.bitcast_convert_type(r[:, :h], jnp.int32)
    hi = lax.bitcast_convert_type(r[:, h:], jnp.int32)
    lo = lax.shift_right_logical(lo + 0x8000, 16)
    hi = (hi + 0x8000) & _HIMASK
    return lo | hi


def _unpack_bf16_pair(w):
    hi = lax.bitcast_convert_type(w, jnp.float32)
    lo = lax.bitcast_convert_type(lax.shift_left(w, 16), jnp.float32)
    return lo, hi


def _sc_gather(table, idx_flat, total, h):
    mesh = plsc.VectorSubcoreMesh(core_axis_name="c", subcore_axis_name="s")
    nbuf = 4
    nfly = 3
    chunk = total // 32
    nw = chunk // GW
    assert chunk % GW == 0 and nw % nbuf == 0

    @functools.partial(
        pl.kernel,
        out_type=jax.ShapeDtypeStruct((total, h), table.dtype),
        mesh=mesh,
        scratch_types=[
            pltpu.VMEM((chunk,), jnp.int32),
            pltpu.VMEM((nbuf, GW, h), jnp.int32),
            pltpu.SemaphoreType.DMA((nbuf,)),
            pltpu.SemaphoreType.DMA((nbuf,)),
        ],
    )
    def k(table_hbm, idx_hbm, out_hbm, idx_v, bufs, sem_g, sem_w):
        wid = lax.axis_index("s") * 2 + lax.axis_index("c")
        base = wid * chunk
        pltpu.sync_copy(idx_hbm.at[0, pl.ds(base, chunk)], idx_v)

        def gather(g, b):
            return pltpu.make_async_copy(
                table_hbm.at[idx_v.at[pl.ds(g * GW, GW)]],
                bufs.at[b], sem_g.at[b])

        def writeback(g, b):
            return pltpu.make_async_copy(
                bufs.at[b], out_hbm.at[pl.ds(base + g * GW, GW)], sem_w.at[b])

        for b in range(nfly):
            gather(b, b).start()

        @pl.loop(0, nw, step=nbuf)
        def _(g0):
            for b in range(nbuf):
                g = g0 + b
                gather(g, b).wait()
                writeback(g, b).start()
                b2 = (b + nfly) % nbuf

                @pl.when(g + nfly < nw)
                def _():
                    @pl.when(g >= nbuf - nfly)
                    def _():
                        writeback(g - (nbuf - nfly), b2).wait()

                    gather(g + nfly, b2).start()

        for b in range(nbuf):
            writeback(nw - nbuf + b, b).wait()

    return k(table, idx_flat)


def _tc_in(x, w_t, b, npad, d):
    n = x.shape[0]
    grid = (npad // BN,)

    def body(x_ref, w_ref, b_ref, o_ref, ot_ref):
        i = pl.program_id(0)
        acc = jnp.dot(x_ref[...], w_ref[...],
                      preferred_element_type=jnp.float32,
                      precision=lax.Precision.HIGHEST)
        r = jnp.maximum(acc + b_ref[...], 0.0)
        row = i * BN + lax.broadcasted_iota(jnp.int32, r.shape, 0)
        r = jnp.where(row < n, r, 0.0)
        o_ref[...] = r
        ot_ref[...] = _pack_bf16_pair(r, d)

    xp = jnp.pad(x, ((0, npad - n), (0, 0)))
    return pl.pallas_call(
        body,
        grid=grid,
        in_specs=[
            pl.BlockSpec((BN, x.shape[1]), lambda i: (i, 0)),
            pl.BlockSpec((x.shape[1], d), lambda i: (0, 0)),
            pl.BlockSpec((1, d), lambda i: (0, 0)),
        ],
        out_specs=[
            pl.BlockSpec((BN, d), lambda i: (i, 0)),
            pl.BlockSpec((BN, d // 2), lambda i: (i, 0)),
        ],
        out_shape=[
            jax.ShapeDtypeStruct((npad, d), jnp.float32),
            jax.ShapeDtypeStruct((npad, d // 2), jnp.int32),
        ],
    )(xp, w_t, b.reshape(1, d))


def _tc_layer(g, in_feats, pw, fc_t, nrows, d, k16, npl, row0):
    grid = (nrows // BN,)
    h = d // 2
    blk0 = row0 // BN

    def body(g_ref, f_ref, pw_ref, fc_ref, o_ref):
        lo, hi = _unpack_bf16_pair(g_ref[...])
        np_ = k16 // npl
        lo4 = lo.reshape(np_, npl, BN, h).sum(axis=0)
        hi4 = hi.reshape(np_, npl, BN, h).sum(axis=0)
        pwv = pw_ref[...]
        acc_lo = jnp.sum(lo4 * pwv[:, None, :h], axis=0)
        acc_hi = jnp.sum(hi4 * pwv[:, None, h:], axis=0)
        r = (jnp.dot(acc_lo, fc_ref[:h, :], preferred_element_type=jnp.float32)
             + jnp.dot(acc_hi, fc_ref[h:, :], preferred_element_type=jnp.float32))
        r = jnp.maximum(r, 0.0)
        feats = ALPHA * f_ref[...] + (1.0 - ALPHA) * r
        o_ref[...] = _pack_bf16_pair(feats, d)

    return pl.pallas_call(
        body,
        grid=grid,
        in_specs=[
            pl.BlockSpec((k16, BN, h), lambda i: (0, i, 0)),
            pl.BlockSpec((BN, d), lambda i: (i + blk0, 0)),
            pl.BlockSpec((npl, d), lambda i: (0, 0)),
            pl.BlockSpec((d, d), lambda i: (0, 0)),
        ],
        out_specs=pl.BlockSpec((BN, h), lambda i: (i, 0)),
        out_shape=jax.ShapeDtypeStruct((nrows, h), jnp.int32),
    )(g, in_feats, pw, fc_t)


def _tc_out(table, w_t, b, npad, d, d_out):
    grid = (npad // BN,)
    h = d // 2

    def body(t_ref, w_ref, b_ref, o_ref):
        lo, hi = _unpack_bf16_pair(t_ref[...])
        acc = (jnp.dot(lo, w_ref[:h, :], preferred_element_type=jnp.float32)
               + jnp.dot(hi, w_ref[h:, :], preferred_element_type=jnp.float32))
        o_ref[...] = acc + b_ref[...]

    return pl.pallas_call(
        body,
        grid=grid,
        in_specs=[
            pl.BlockSpec((BN, h), lambda i: (i, 0)),
            pl.BlockSpec((d, d_out), lambda i: (0, 0)),
            pl.BlockSpec((1, d_out), lambda i: (0, 0)),
        ],
        out_specs=pl.BlockSpec((BN, d_out), lambda i: (i, 0)),
        out_shape=jax.ShapeDtypeStruct((npad, d_out), jnp.float32),
    )(table, w_t, b.reshape(1, d_out))


def kernel(input_x, paths, W_in, b_in, W_out, b_out, path_weight, fc_w):
    n, in_dim = input_x.shape
    p, _, pl_len = paths.shape
    hidden = W_in.shape[0]
    out_dim = W_out.shape[0]
    num_layers = fc_w.shape[0]
    k16 = p * pl_len

    npad = ((n + 1023) // 1024) * 1024
    assert npad % BN == 0
    nchunk = 1
    half = npad // nchunk
    thalf = k16 * half
    assert thalf % (GW * 32) == 0 and half % BN == 0

    idx = paths.transpose(0, 2, 1).reshape(k16, n)
    idx = jnp.pad(idx, ((0, 0), (0, npad - n)))

    pw_all = path_weight[:, 0, :, :] / p

    idx_halves = [idx[:, c * half:(c + 1) * half].reshape(1, thalf)
                  for c in range(nchunk)]

    in_feats, table = _tc_in(input_x, W_in.T, b_in, npad, hidden)
    for l in range(num_layers):
        parts = []
        for c in range(nchunk):
            g = _sc_gather(table, idx_halves[c], thalf, hidden // 2)
            g = g.reshape(k16, half, hidden // 2)
            parts.append(_tc_layer(g, in_feats, pw_all[l], fc_w[l].T, half,
                                   hidden, k16, pl_len, c * half))
        table = jnp.concatenate(parts, axis=0)
    out = _tc_out(table, W_out.T, b_out, npad, hidden, out_dim)
    return out[:n]


# speedup vs baseline: 2.0619x; 1.0333x over previous
import functools

import jax
import jax.numpy as jnp
from jax import lax
from jax.experimental import pallas as pl
from jax.experimental.pallas import tpu as pltpu
from jax.experimental.pallas import tpu_sc as plsc

ALPHA = 0.1
GW = 128
BN = 512

_HIMASK = -65536


def _pack_bf16_pair(r, d):
    h = d // 2
    lo = lax.bitcast_convert_type(r[:, :h], jnp.int32)
    hi = lax.bitcast_convert_type(r[:, h:], jnp.int32)
    lo = lax.shift_right_logical(lo + 0x8000, 16)
    hi = (hi + 0x8000) & _HIMASK
    return lo | hi


def _unpack_bf16_pair(w):
    hi = lax.bitcast_convert_type(w, jnp.float32)
    lo = lax.bitcast_convert_type(lax.shift_left(w, 16), jnp.float32)
    return lo, hi


def _sc_gather(table, idx_flat, total, h):
    mesh = plsc.VectorSubcoreMesh(core_axis_name="c", subcore_axis_name="s")
    nbuf = 4
    nfly = 3
    chunk = total // 32
    nw = chunk // GW
    assert chunk % GW == 0 and nw % nbuf == 0

    @functools.partial(
        pl.kernel,
        out_type=jax.ShapeDtypeStruct((total, h), table.dtype),
        mesh=mesh,
        scratch_types=[
            pltpu.VMEM((chunk,), jnp.int32),
            pltpu.VMEM((nbuf, GW, h), jnp.int32),
            pltpu.SemaphoreType.DMA((nbuf,)),
            pltpu.SemaphoreType.DMA((nbuf,)),
        ],
    )
    def k(table_hbm, idx_hbm, out_hbm, idx_v, bufs, sem_g, sem_w):
        wid = lax.axis_index("s") * 2 + lax.axis_index("c")
        base = wid * chunk
        pltpu.sync_copy(idx_hbm.at[0, pl.ds(base, chunk)], idx_v)

        def gather(g, b):
            return pltpu.make_async_copy(
                table_hbm.at[idx_v.at[pl.ds(g * GW, GW)]],
                bufs.at[b], sem_g.at[b])

        def writeback(g, b):
            return pltpu.make_async_copy(
                bufs.at[b], out_hbm.at[pl.ds(base + g * GW, GW)], sem_w.at[b])

        for b in range(nfly):
            gather(b, b).start()

        @pl.loop(0, nw, step=nbuf)
        def _(g0):
            for b in range(nbuf):
                g = g0 + b
                gather(g, b).wait()
                writeback(g, b).start()
                b2 = (b + nfly) % nbuf

                @pl.when(g + nfly < nw)
                def _():
                    @pl.when(g >= nbuf - nfly)
                    def _():
                        writeback(g - (nbuf - nfly), b2).wait()

                    gather(g + nfly, b2).start()

        for b in range(nbuf):
            writeback(nw - nbuf + b, b).wait()

    return k(table, idx_flat)


def _tc_in(x, w_t, b, npad, d):
    n = x.shape[0]
    grid = (npad // BN,)

    def body(x_ref, w_ref, b_ref, o_ref, ot_ref):
        i = pl.program_id(0)
        acc = jnp.dot(x_ref[...], w_ref[...],
                      preferred_element_type=jnp.float32,
                      precision=lax.Precision.HIGHEST)
        r = jnp.maximum(acc + b_ref[...], 0.0)
        row = i * BN + lax.broadcasted_iota(jnp.int32, r.shape, 0)
        r = jnp.where(row < n, r, 0.0)
        o_ref[...] = r
        ot_ref[...] = _pack_bf16_pair(r, d)

    xp = jnp.pad(x, ((0, npad - n), (0, 0)))
    return pl.pallas_call(
        body,
        grid=grid,
        in_specs=[
            pl.BlockSpec((BN, x.shape[1]), lambda i: (i, 0)),
            pl.BlockSpec((x.shape[1], d), lambda i: (0, 0)),
            pl.BlockSpec((1, d), lambda i: (0, 0)),
        ],
        out_specs=[
            pl.BlockSpec((BN, d), lambda i: (i, 0)),
            pl.BlockSpec((BN, d // 2), lambda i: (i, 0)),
        ],
        out_shape=[
            jax.ShapeDtypeStruct((npad, d), jnp.float32),
            jax.ShapeDtypeStruct((npad, d // 2), jnp.int32),
        ],
    )(xp, w_t, b.reshape(1, d))


def _tc_layer(g, in_feats, pw, fc_t, nrows, d, k16, npl, row0):
    grid = (nrows // BN,)
    h = d // 2
    blk0 = row0 // BN

    def body(g_ref, f_ref, pw_ref, fc_ref, o_ref):
        lo, hi = _unpack_bf16_pair(g_ref[...])
        np_ = k16 // npl
        lo4 = lo.reshape(np_, npl, BN, h).sum(axis=0)
        hi4 = hi.reshape(np_, npl, BN, h).sum(axis=0)
        pwv = pw_ref[...]
        acc_lo = jnp.sum(lo4 * pwv[:, None, :h], axis=0)
        acc_hi = jnp.sum(hi4 * pwv[:, None, h:], axis=0)
        r = (jnp.dot(acc_lo, fc_ref[:h, :], preferred_element_type=jnp.float32)
             + jnp.dot(acc_hi, fc_ref[h:, :], preferred_element_type=jnp.float32))
        r = jnp.maximum(r, 0.0)
        feats = ALPHA * f_ref[...] + (1.0 - ALPHA) * r
        o_ref[...] = _pack_bf16_pair(feats, d)

    return pl.pallas_call(
        body,
        grid=grid,
        in_specs=[
            pl.BlockSpec((k16, BN, h), lambda i: (0, i, 0)),
            pl.BlockSpec((BN, d), lambda i: (i + blk0, 0)),
            pl.BlockSpec((npl, d), lambda i: (0, 0)),
            pl.BlockSpec((d, d), lambda i: (0, 0)),
        ],
        out_specs=pl.BlockSpec((BN, h), lambda i: (i, 0)),
        out_shape=jax.ShapeDtypeStruct((nrows, h), jnp.int32),
    )(g, in_feats, pw, fc_t)


def _tc_layer_out(g, in_feats, pw, fc_t, w_t, b, nrows, d, k16, npl, d_out):
    grid = (nrows // BN,)
    h = d // 2

    def body(g_ref, f_ref, pw_ref, fc_ref, w_ref, b_ref, o_ref):
        lo, hi = _unpack_bf16_pair(g_ref[...])
        np_ = k16 // npl
        lo4 = lo.reshape(np_, npl, BN, h).sum(axis=0)
        hi4 = hi.reshape(np_, npl, BN, h).sum(axis=0)
        pwv = pw_ref[...]
        acc_lo = jnp.sum(lo4 * pwv[:, None, :h], axis=0)
        acc_hi = jnp.sum(hi4 * pwv[:, None, h:], axis=0)
        r = (jnp.dot(acc_lo, fc_ref[:h, :], preferred_element_type=jnp.float32)
             + jnp.dot(acc_hi, fc_ref[h:, :], preferred_element_type=jnp.float32))
        r = jnp.maximum(r, 0.0)
        feats = ALPHA * f_ref[...] + (1.0 - ALPHA) * r
        acc = jnp.dot(feats, w_ref[...], preferred_element_type=jnp.float32)
        o_ref[...] = acc + b_ref[...]

    return pl.pallas_call(
        body,
        grid=grid,
        in_specs=[
            pl.BlockSpec((k16, BN, h), lambda i: (0, i, 0)),
            pl.BlockSpec((BN, d), lambda i: (i, 0)),
            pl.BlockSpec((npl, d), lambda i: (0, 0)),
            pl.BlockSpec((d, d), lambda i: (0, 0)),
            pl.BlockSpec((d, d_out), lambda i: (0, 0)),
            pl.BlockSpec((1, d_out), lambda i: (0, 0)),
        ],
        out_specs=pl.BlockSpec((BN, d_out), lambda i: (i, 0)),
        out_shape=jax.ShapeDtypeStruct((nrows, d_out), jnp.float32),
    )(g, in_feats, pw, fc_t, w_t, b.reshape(1, d_out))


def kernel(input_x, paths, W_in, b_in, W_out, b_out, path_weight, fc_w):
    n, in_dim = input_x.shape
    p, _, pl_len = paths.shape
    hidden = W_in.shape[0]
    out_dim = W_out.shape[0]
    num_layers = fc_w.shape[0]
    k16 = p * pl_len

    npad = ((n + 1023) // 1024) * 1024
    assert npad % BN == 0
    nchunk = 1
    half = npad // nchunk
    thalf = k16 * half
    assert thalf % (GW * 32) == 0 and half % BN == 0

    idx = paths.transpose(0, 2, 1).reshape(k16, n)
    idx = jnp.pad(idx, ((0, 0), (0, npad - n)))

    pw_all = path_weight[:, 0, :, :] / p

    idx_halves = [idx[:, c * half:(c + 1) * half].reshape(1, thalf)
                  for c in range(nchunk)]

    in_feats, table = _tc_in(input_x, W_in.T, b_in, npad, hidden)
    for l in range(num_layers):
        g = _sc_gather(table, idx_halves[0], thalf, hidden // 2)
        g = g.reshape(k16, npad, hidden // 2)
        if l < num_layers - 1:
            table = _tc_layer(g, in_feats, pw_all[l], fc_w[l].T, npad,
                              hidden, k16, pl_len, 0)
        else:
            out = _tc_layer_out(g, in_feats, pw_all[l], fc_w[l].T, W_out.T,
                                b_out, npad, hidden, k16, pl_len, out_dim)
    return out[:n]
